# Initial kernel scaffold; baseline (speedup 1.0000x reference)
import jax
import jax.numpy as jnp
from jax.experimental import pallas as pl


def kernel(x, gate_W, gate_b, expert_W, expert_b):
    raise NotImplementedError("write your pallas kernel here")


# speedup vs baseline: 4.8678x; 4.8678x over previous
import jax
import jax.numpy as jnp
from jax.experimental import pallas as pl
from jax.experimental.pallas import tpu as pltpu

_E = 64
_K = 8
_TB = 512


def _moe_gate_kernel(x_ref, w_ref, b_ref, o_ref):
    x = x_ref[...]
    w = w_ref[...]
    y = jnp.dot(x, w, preferred_element_type=jnp.float32) + b_ref[...]
    logits = y[:, :_E]
    expert = y[:, _E:]

    m = jnp.max(logits, axis=1, keepdims=True)
    p = jnp.exp(logits - m)
    z = jnp.sum(p, axis=1, keepdims=True)

    iota = jax.lax.broadcasted_iota(jnp.int32, logits.shape, 1)
    mask = jnp.zeros(logits.shape, dtype=jnp.bool_)
    work = logits
    for _ in range(_K):
        cm = jnp.max(work, axis=1, keepdims=True)
        first = jnp.min(jnp.where(work == cm, iota, _E), axis=1, keepdims=True)
        sel = iota == first
        mask = jnp.logical_or(mask, sel)
        work = jnp.where(sel, -jnp.inf, work)

    o_ref[...] = jnp.sum(jnp.where(mask, p, 0.0) * expert, axis=1, keepdims=True) / z


def kernel(x, gate_W, gate_b, expert_W, expert_b):
    b, d = x.shape
    w = jnp.concatenate([gate_W, expert_W.T], axis=1)
    bias = jnp.concatenate([gate_b, expert_b]).reshape(1, 2 * _E)
    grid = (b // _TB,)
    return pl.pallas_call(
        _moe_gate_kernel,
        grid=grid,
        in_specs=[
            pl.BlockSpec((_TB, d), lambda i: (i, 0)),
            pl.BlockSpec((d, 2 * _E), lambda i: (0, 0)),
            pl.BlockSpec((1, 2 * _E), lambda i: (0, 0)),
        ],
        out_specs=pl.BlockSpec((_TB, 1), lambda i: (i, 0)),
        out_shape=jax.ShapeDtypeStruct((b, 1), jnp.float32),
        compiler_params=pltpu.CompilerParams(
            dimension_semantics=("arbitrary",),
        ),
    )(x, w, bias)


# speedup vs baseline: 6.2356x; 1.2810x over previous
import jax
import jax.numpy as jnp
from jax.experimental import pallas as pl
from jax.experimental.pallas import tpu as pltpu

_E = 64
_K = 8
_TB = 512


def _moe_gate_kernel(x_ref, w_ref, b_ref, o_ref):
    x = x_ref[...]
    w = w_ref[...]
    y = jnp.dot(x, w, preferred_element_type=jnp.float32) + b_ref[...]
    logits = y[:, :_E]
    expert = y[:, _E:]

    m = jnp.max(logits, axis=1, keepdims=True)
    p = jnp.exp(logits - m)
    z = jnp.sum(p, axis=1, keepdims=True)

    work = logits
    for _ in range(_K - 1):
        cm = jnp.max(work, axis=1, keepdims=True)
        work = jnp.where(work == cm, -jnp.inf, work)
    thresh = jnp.max(work, axis=1, keepdims=True)
    keep = logits >= thresh

    o_ref[...] = jnp.sum(jnp.where(keep, p, 0.0) * expert, axis=1, keepdims=True) / z


def kernel(x, gate_W, gate_b, expert_W, expert_b):
    b, d = x.shape
    w = jnp.concatenate([gate_W, expert_W.T], axis=1)
    bias = jnp.concatenate([gate_b, expert_b]).reshape(1, 2 * _E)
    grid = (b // _TB,)
    return pl.pallas_call(
        _moe_gate_kernel,
        grid=grid,
        in_specs=[
            pl.BlockSpec((_TB, d), lambda i: (i, 0)),
            pl.BlockSpec((d, 2 * _E), lambda i: (0, 0)),
            pl.BlockSpec((1, 2 * _E), lambda i: (0, 0)),
        ],
        out_specs=pl.BlockSpec((_TB, 1), lambda i: (i, 0)),
        out_shape=jax.ShapeDtypeStruct((b, 1), jnp.float32),
        compiler_params=pltpu.CompilerParams(
            dimension_semantics=("arbitrary",),
        ),
    )(x, w, bias)
